# baseline (speedup 1.0000x reference)
import functools

import jax
import jax.numpy as jnp
from jax.experimental import pallas as pl
from jax.experimental.pallas import tpu as pltpu
from jax.experimental.pallas import tpu_sc as plsc

N = 100000
D = 128
VP = 40
NC = 2
CS = 51200
S = NC * CS
MC = 4 * CS
BN = 400
CB = CS // BN
HALF_BLOCK = (N // 2) // BN
GW = 64
NSTR = 4
GB = GW * NSTR

_CHUNK_REAL = [min(CS, N - c * CS) for c in range(NC)]


def _sc_gather(table, idx2d, m_rows):
    mesh = plsc.VectorSubcoreMesh(core_axis_name="c", subcore_axis_name="s")

    @functools.partial(
        pl.kernel,
        out_type=jax.ShapeDtypeStruct((m_rows, D), jnp.float32),
        mesh=mesh,
        scratch_types=[pltpu.SemaphoreType.DMA],
    )
    def gather_kernel(tab_hbm, idx_hbm, out_hbm, gsem):
        def body(i_vmem, o_vmem):
            for j in range(NSTR):
                pltpu.async_copy(
                    tab_hbm.at[i_vmem.at[0, pl.ds(j * GW, GW)]],
                    o_vmem.at[pl.ds(j * GW, GW), :],
                    gsem,
                )
            for j in range(NSTR):
                pltpu.make_async_copy(
                    tab_hbm.at[i_vmem.at[0, pl.ds(j * GW, GW)]],
                    o_vmem.at[pl.ds(j * GW, GW), :],
                    gsem,
                ).wait()

        pltpu.emit_pipeline(
            body,
            grid=(m_rows // GB,),
            in_specs=[pl.BlockSpec((1, GB), lambda i: (0, i))],
            out_specs=[pl.BlockSpec((GB, D), lambda i: (i, 0))],
            core_axis_name=("c", "s"),
            dimension_semantics=(pltpu.PARALLEL,),
        )(idx_hbm, out_hbm)

    return gather_kernel(table, idx2d)


def _tc_embed(nid3d, embp):
    def body(ids_ref, emb_ref, out):
        ids = ids_ref[0, 0, :]
        iota = jax.lax.broadcasted_iota(jnp.int32, (BN, VP), 1)
        oh = (ids.reshape(BN, 1) == iota).astype(jnp.float32)
        out[...] = jnp.dot(oh, emb_ref[...], preferred_element_type=jnp.float32)

    return pl.pallas_call(
        body,
        grid=(N // BN,),
        in_specs=[
            pl.BlockSpec((1, 1, BN), lambda i: (i, 0, 0)),
            pl.BlockSpec((VP, D), lambda i: (0, 0)),
        ],
        out_specs=pl.BlockSpec((BN, D), lambda i: (i, 0)),
        out_shape=jax.ShapeDtypeStruct((N, D), jnp.float32),
    )(nid3d, embp)


def _mlp_block(m0, m1, m2, m3, w0, b0r, w1, b1r):
    bf = jnp.bfloat16
    x = jnp.dot(m0[...].astype(bf), w0[0:128, :], preferred_element_type=jnp.float32)
    x = x + jnp.dot(m1[...].astype(bf), w0[128:256, :], preferred_element_type=jnp.float32)
    x = x + jnp.dot(m2[...].astype(bf), w0[256:384, :], preferred_element_type=jnp.float32)
    x = x + jnp.dot(m3[...].astype(bf), w0[384:512, :], preferred_element_type=jnp.float32)
    x = jnp.maximum(x + b0r[...], 0.0).astype(bf)
    return jnp.dot(x, w1[...], preferred_element_type=jnp.float32) + b1r[...]


def _chunk_in_specs(grid_c):
    del grid_c
    return [
        pl.BlockSpec((BN, D), (lambda i, k=k: (k * CB + i, 0)))
        for k in range(4)
    ] + [
        pl.BlockSpec((4 * D, D), lambda i: (0, 0)),
        pl.BlockSpec((1, D), lambda i: (0, 0)),
        pl.BlockSpec((D, D), lambda i: (0, 0)),
        pl.BlockSpec((1, D), lambda i: (0, 0)),
    ]


def _tc_layer_chunk(c, msgs_c, W0, b0_2d, W1, b1_2d, hbuf):
    grid_c = _CHUNK_REAL[c] // BN

    def body(m0, m1, m2, m3, w0, b0r, w1, b1r, _hb, out):
        out[...] = _mlp_block(m0, m1, m2, m3, w0, b0r, w1, b1r)

    return pl.pallas_call(
        body,
        grid=(grid_c,),
        in_specs=_chunk_in_specs(grid_c) + [
            pl.BlockSpec((BN, D), (lambda i: (c * CB + i, 0))),
        ],
        out_specs=pl.BlockSpec((BN, D), (lambda i: (c * CB + i, 0))),
        out_shape=jax.ShapeDtypeStruct((N, D), jnp.float32),
        input_output_aliases={8: 0},
    )(msgs_c, msgs_c, msgs_c, msgs_c, W0, b0_2d, W1, b1_2d, hbuf)


def _tc_final_chunk(c, msgs_c, W0, b0_2d, W1, b1_2d):
    grid_c = _CHUNK_REAL[c] // BN

    def body(m0, m1, m2, m3, w0, b0r, w1, b1r, out):
        i = pl.program_id(0)
        h = _mlp_block(m0, m1, m2, m3, w0, b0r, w1, b1r)
        colsum = jnp.sum(h, axis=0, keepdims=True)

        @pl.when(i == 0)
        def _():
            out[...] = jnp.zeros_like(out)

        @pl.when(c * CB + i < HALF_BLOCK)
        def _():
            out[0:1, :] = out[0:1, :] + colsum

        @pl.when(c * CB + i >= HALF_BLOCK)
        def _():
            out[1:2, :] = out[1:2, :] + colsum

    return pl.pallas_call(
        body,
        grid=(grid_c,),
        in_specs=_chunk_in_specs(grid_c),
        out_specs=pl.BlockSpec((2, D), lambda i: (0, 0)),
        out_shape=jax.ShapeDtypeStruct((2, D), jnp.float32),
    )(msgs_c, msgs_c, msgs_c, msgs_c, W0, b0_2d, W1, b1_2d)


def kernel(node_ids, neighbor_idx, emb,
           l0_W0, l0_b0, l0_W1, l0_b1,
           l1_W0, l1_b0, l1_W1, l1_b1,
           l2_W0, l2_b0, l2_W1, l2_b1,
           output_bias):
    nbrT = jnp.transpose(neighbor_idx.astype(jnp.int32))
    nbrP = jnp.pad(nbrT, ((0, 0), (0, S - N)))
    idxc = jnp.transpose(nbrP.reshape(4, NC, CS), (1, 0, 2))
    idxc = idxc.reshape(NC, 1, MC)
    nid3d = node_ids.astype(jnp.int32).reshape(N // BN, 1, BN)
    embp = jnp.pad(emb, ((0, VP - emb.shape[0]), (0, 0)))

    params = [
        (l0_W0.astype(jnp.bfloat16), l0_b0.reshape(1, D),
         l0_W1.astype(jnp.bfloat16), l0_b1.reshape(1, D)),
        (l1_W0.astype(jnp.bfloat16), l1_b0.reshape(1, D),
         l1_W1.astype(jnp.bfloat16), l1_b1.reshape(1, D)),
        (l2_W0.astype(jnp.bfloat16), l2_b0.reshape(1, D),
         l2_W1.astype(jnp.bfloat16), l2_b1.reshape(1, D)),
    ]

    h = _tc_embed(nid3d, embp)
    for li, (W0, b0r, W1, b1r) in enumerate(params):
        msgs = [_sc_gather(h, idxc[c], MC) for c in range(NC)]
        if li < 2:
            hbuf = jnp.zeros((N, D), jnp.float32)
            for c in range(NC):
                hbuf = _tc_layer_chunk(c, msgs[c], W0, b0r, W1, b1r, hbuf)
            h = hbuf
        else:
            partials = [_tc_final_chunk(c, msgs[c], W0, b0r, W1, b1r)
                        for c in range(NC)]
    sums = sum(partials)
    half = jnp.float32(N // 2)
    logit = jnp.sum(sums[0] * sums[1], keepdims=True) / (half * half)
    return logit + output_bias


# speedup vs baseline: 1.0308x; 1.0308x over previous
import functools

import jax
import jax.numpy as jnp
from jax.experimental import pallas as pl
from jax.experimental.pallas import tpu as pltpu
from jax.experimental.pallas import tpu_sc as plsc

N = 100000
D = 128
VP = 40
NC = 16
CS = 6400
S = NC * CS
MC = 4 * CS
BN = 400
CB = CS // BN
HALF_BLOCK = (N // 2) // BN
GW = 64
NSTR = 4
GB = GW * NSTR

_CHUNK_REAL = [min(CS, N - c * CS) for c in range(NC)]


def _sc_gather(table, idx2d, m_rows):
    mesh = plsc.VectorSubcoreMesh(core_axis_name="c", subcore_axis_name="s")

    @functools.partial(
        pl.kernel,
        out_type=jax.ShapeDtypeStruct((m_rows, D), jnp.float32),
        mesh=mesh,
        scratch_types=[pltpu.SemaphoreType.DMA],
    )
    def gather_kernel(tab_hbm, idx_hbm, out_hbm, gsem):
        def body(i_vmem, o_vmem):
            for j in range(NSTR):
                pltpu.async_copy(
                    tab_hbm.at[i_vmem.at[0, pl.ds(j * GW, GW)]],
                    o_vmem.at[pl.ds(j * GW, GW), :],
                    gsem,
                )
            for j in range(NSTR):
                pltpu.make_async_copy(
                    tab_hbm.at[i_vmem.at[0, pl.ds(j * GW, GW)]],
                    o_vmem.at[pl.ds(j * GW, GW), :],
                    gsem,
                ).wait()

        pltpu.emit_pipeline(
            body,
            grid=(m_rows // GB,),
            in_specs=[pl.BlockSpec((1, GB), lambda i: (0, i))],
            out_specs=[pl.BlockSpec((GB, D), lambda i: (i, 0))],
            core_axis_name=("c", "s"),
            dimension_semantics=(pltpu.PARALLEL,),
        )(idx_hbm, out_hbm)

    return gather_kernel(table, idx2d)


def _tc_embed(nid3d, embp):
    def body(ids_ref, emb_ref, out):
        ids = ids_ref[0, 0, :]
        iota = jax.lax.broadcasted_iota(jnp.int32, (BN, VP), 1)
        oh = (ids.reshape(BN, 1) == iota).astype(jnp.float32)
        out[...] = jnp.dot(oh, emb_ref[...], preferred_element_type=jnp.float32)

    return pl.pallas_call(
        body,
        grid=(N // BN,),
        in_specs=[
            pl.BlockSpec((1, 1, BN), lambda i: (i, 0, 0)),
            pl.BlockSpec((VP, D), lambda i: (0, 0)),
        ],
        out_specs=pl.BlockSpec((BN, D), lambda i: (i, 0)),
        out_shape=jax.ShapeDtypeStruct((N, D), jnp.float32),
    )(nid3d, embp)


def _mlp_block(m0, m1, m2, m3, w0, b0r, w1, b1r):
    bf = jnp.bfloat16
    x = jnp.dot(m0[...].astype(bf), w0[0:128, :], preferred_element_type=jnp.float32)
    x = x + jnp.dot(m1[...].astype(bf), w0[128:256, :], preferred_element_type=jnp.float32)
    x = x + jnp.dot(m2[...].astype(bf), w0[256:384, :], preferred_element_type=jnp.float32)
    x = x + jnp.dot(m3[...].astype(bf), w0[384:512, :], preferred_element_type=jnp.float32)
    x = jnp.maximum(x + b0r[...], 0.0).astype(bf)
    return jnp.dot(x, w1[...], preferred_element_type=jnp.float32) + b1r[...]


def _chunk_in_specs(grid_c):
    del grid_c
    return [
        pl.BlockSpec((BN, D), (lambda i, k=k: (k * CB + i, 0)))
        for k in range(4)
    ] + [
        pl.BlockSpec((4 * D, D), lambda i: (0, 0)),
        pl.BlockSpec((1, D), lambda i: (0, 0)),
        pl.BlockSpec((D, D), lambda i: (0, 0)),
        pl.BlockSpec((1, D), lambda i: (0, 0)),
    ]


def _tc_layer_chunk(c, msgs_c, W0, b0_2d, W1, b1_2d, hbuf):
    grid_c = _CHUNK_REAL[c] // BN

    def body(m0, m1, m2, m3, w0, b0r, w1, b1r, _hb, out):
        out[...] = _mlp_block(m0, m1, m2, m3, w0, b0r, w1, b1r)

    return pl.pallas_call(
        body,
        grid=(grid_c,),
        in_specs=_chunk_in_specs(grid_c) + [
            pl.BlockSpec((BN, D), (lambda i: (c * CB + i, 0))),
        ],
        out_specs=pl.BlockSpec((BN, D), (lambda i: (c * CB + i, 0))),
        out_shape=jax.ShapeDtypeStruct((N, D), jnp.float32),
        input_output_aliases={8: 0},
    )(msgs_c, msgs_c, msgs_c, msgs_c, W0, b0_2d, W1, b1_2d, hbuf)


def _tc_final_chunk(c, msgs_c, W0, b0_2d, W1, b1_2d):
    grid_c = _CHUNK_REAL[c] // BN

    def body(m0, m1, m2, m3, w0, b0r, w1, b1r, out):
        i = pl.program_id(0)
        h = _mlp_block(m0, m1, m2, m3, w0, b0r, w1, b1r)
        colsum = jnp.sum(h, axis=0, keepdims=True)

        @pl.when(i == 0)
        def _():
            out[...] = jnp.zeros_like(out)

        @pl.when(c * CB + i < HALF_BLOCK)
        def _():
            out[0:1, :] = out[0:1, :] + colsum

        @pl.when(c * CB + i >= HALF_BLOCK)
        def _():
            out[1:2, :] = out[1:2, :] + colsum

    return pl.pallas_call(
        body,
        grid=(grid_c,),
        in_specs=_chunk_in_specs(grid_c),
        out_specs=pl.BlockSpec((2, D), lambda i: (0, 0)),
        out_shape=jax.ShapeDtypeStruct((2, D), jnp.float32),
    )(msgs_c, msgs_c, msgs_c, msgs_c, W0, b0_2d, W1, b1_2d)


def kernel(node_ids, neighbor_idx, emb,
           l0_W0, l0_b0, l0_W1, l0_b1,
           l1_W0, l1_b0, l1_W1, l1_b1,
           l2_W0, l2_b0, l2_W1, l2_b1,
           output_bias):
    nbrT = jnp.transpose(neighbor_idx.astype(jnp.int32))
    nbrP = jnp.pad(nbrT, ((0, 0), (0, S - N)))
    idxc = jnp.transpose(nbrP.reshape(4, NC, CS), (1, 0, 2))
    idxc = idxc.reshape(NC, 1, MC)
    nid3d = node_ids.astype(jnp.int32).reshape(N // BN, 1, BN)
    embp = jnp.pad(emb, ((0, VP - emb.shape[0]), (0, 0)))

    params = [
        (l0_W0.astype(jnp.bfloat16), l0_b0.reshape(1, D),
         l0_W1.astype(jnp.bfloat16), l0_b1.reshape(1, D)),
        (l1_W0.astype(jnp.bfloat16), l1_b0.reshape(1, D),
         l1_W1.astype(jnp.bfloat16), l1_b1.reshape(1, D)),
        (l2_W0.astype(jnp.bfloat16), l2_b0.reshape(1, D),
         l2_W1.astype(jnp.bfloat16), l2_b1.reshape(1, D)),
    ]

    h = _tc_embed(nid3d, embp)
    for li, (W0, b0r, W1, b1r) in enumerate(params):
        msgs = [_sc_gather(h, idxc[c], MC) for c in range(NC)]
        if li < 2:
            hbuf = jnp.zeros((N, D), jnp.float32)
            for c in range(NC):
                hbuf = _tc_layer_chunk(c, msgs[c], W0, b0r, W1, b1r, hbuf)
            h = hbuf
        else:
            partials = [_tc_final_chunk(c, msgs[c], W0, b0r, W1, b1r)
                        for c in range(NC)]
    sums = sum(partials)
    half = jnp.float32(N // 2)
    logit = jnp.sum(sums[0] * sums[1], keepdims=True) / (half * half)
    return logit + output_bias


# speedup vs baseline: 1.0543x; 1.0228x over previous
import functools

import jax
import jax.numpy as jnp
from jax.experimental import pallas as pl
from jax.experimental.pallas import tpu as pltpu
from jax.experimental.pallas import tpu_sc as plsc

N = 100000
D = 128
VP = 40
NC = 8
CS = 12800
S = NC * CS
MC = 4 * CS
BN = 400
CB = CS // BN
HALF_BLOCK = (N // 2) // BN
GW = 256
NSTR = 1
GB = GW * NSTR

_CHUNK_REAL = [min(CS, N - c * CS) for c in range(NC)]


def _sc_gather(table, idx2d, m_rows):
    mesh = plsc.VectorSubcoreMesh(core_axis_name="c", subcore_axis_name="s")

    @functools.partial(
        pl.kernel,
        out_type=jax.ShapeDtypeStruct((m_rows, D), jnp.float32),
        mesh=mesh,
        scratch_types=[pltpu.SemaphoreType.DMA],
    )
    def gather_kernel(tab_hbm, idx_hbm, out_hbm, gsem):
        def body(i_vmem, o_vmem):
            for j in range(NSTR):
                pltpu.async_copy(
                    tab_hbm.at[i_vmem.at[0, pl.ds(j * GW, GW)]],
                    o_vmem.at[pl.ds(j * GW, GW), :],
                    gsem,
                )
            for j in range(NSTR):
                pltpu.make_async_copy(
                    tab_hbm.at[i_vmem.at[0, pl.ds(j * GW, GW)]],
                    o_vmem.at[pl.ds(j * GW, GW), :],
                    gsem,
                ).wait()

        pltpu.emit_pipeline(
            body,
            grid=(m_rows // GB,),
            in_specs=[pl.BlockSpec((1, GB), lambda i: (0, i))],
            out_specs=[pl.BlockSpec((GB, D), lambda i: (i, 0))],
            core_axis_name=("c", "s"),
            dimension_semantics=(pltpu.PARALLEL,),
        )(idx_hbm, out_hbm)

    return gather_kernel(table, idx2d)


def _tc_embed(nid3d, embp):
    def body(ids_ref, emb_ref, out):
        ids = ids_ref[0, 0, :]
        iota = jax.lax.broadcasted_iota(jnp.int32, (BN, VP), 1)
        oh = (ids.reshape(BN, 1) == iota).astype(jnp.float32)
        out[...] = jnp.dot(oh, emb_ref[...], preferred_element_type=jnp.float32)

    return pl.pallas_call(
        body,
        grid=(N // BN,),
        in_specs=[
            pl.BlockSpec((1, 1, BN), lambda i: (i, 0, 0)),
            pl.BlockSpec((VP, D), lambda i: (0, 0)),
        ],
        out_specs=pl.BlockSpec((BN, D), lambda i: (i, 0)),
        out_shape=jax.ShapeDtypeStruct((N, D), jnp.float32),
    )(nid3d, embp)


def _mlp_block(m0, m1, m2, m3, w0, b0r, w1, b1r):
    bf = jnp.bfloat16
    x = jnp.dot(m0[...].astype(bf), w0[0:128, :], preferred_element_type=jnp.float32)
    x = x + jnp.dot(m1[...].astype(bf), w0[128:256, :], preferred_element_type=jnp.float32)
    x = x + jnp.dot(m2[...].astype(bf), w0[256:384, :], preferred_element_type=jnp.float32)
    x = x + jnp.dot(m3[...].astype(bf), w0[384:512, :], preferred_element_type=jnp.float32)
    x = jnp.maximum(x + b0r[...], 0.0).astype(bf)
    return jnp.dot(x, w1[...], preferred_element_type=jnp.float32) + b1r[...]


def _chunk_in_specs(grid_c):
    del grid_c
    return [
        pl.BlockSpec((BN, D), (lambda i, k=k: (k * CB + i, 0)))
        for k in range(4)
    ] + [
        pl.BlockSpec((4 * D, D), lambda i: (0, 0)),
        pl.BlockSpec((1, D), lambda i: (0, 0)),
        pl.BlockSpec((D, D), lambda i: (0, 0)),
        pl.BlockSpec((1, D), lambda i: (0, 0)),
    ]


def _tc_layer_chunk(c, msgs_c, W0, b0_2d, W1, b1_2d, hbuf):
    grid_c = _CHUNK_REAL[c] // BN

    def body(m0, m1, m2, m3, w0, b0r, w1, b1r, _hb, out):
        out[...] = _mlp_block(m0, m1, m2, m3, w0, b0r, w1, b1r)

    return pl.pallas_call(
        body,
        grid=(grid_c,),
        in_specs=_chunk_in_specs(grid_c) + [
            pl.BlockSpec((BN, D), (lambda i: (c * CB + i, 0))),
        ],
        out_specs=pl.BlockSpec((BN, D), (lambda i: (c * CB + i, 0))),
        out_shape=jax.ShapeDtypeStruct((N, D), jnp.float32),
        input_output_aliases={8: 0},
    )(msgs_c, msgs_c, msgs_c, msgs_c, W0, b0_2d, W1, b1_2d, hbuf)


def _tc_final_chunk(c, msgs_c, W0, b0_2d, W1, b1_2d):
    grid_c = _CHUNK_REAL[c] // BN

    def body(m0, m1, m2, m3, w0, b0r, w1, b1r, out):
        i = pl.program_id(0)
        h = _mlp_block(m0, m1, m2, m3, w0, b0r, w1, b1r)
        colsum = jnp.sum(h, axis=0, keepdims=True)

        @pl.when(i == 0)
        def _():
            out[...] = jnp.zeros_like(out)

        @pl.when(c * CB + i < HALF_BLOCK)
        def _():
            out[0:1, :] = out[0:1, :] + colsum

        @pl.when(c * CB + i >= HALF_BLOCK)
        def _():
            out[1:2, :] = out[1:2, :] + colsum

    return pl.pallas_call(
        body,
        grid=(grid_c,),
        in_specs=_chunk_in_specs(grid_c),
        out_specs=pl.BlockSpec((2, D), lambda i: (0, 0)),
        out_shape=jax.ShapeDtypeStruct((2, D), jnp.float32),
    )(msgs_c, msgs_c, msgs_c, msgs_c, W0, b0_2d, W1, b1_2d)


def kernel(node_ids, neighbor_idx, emb,
           l0_W0, l0_b0, l0_W1, l0_b1,
           l1_W0, l1_b0, l1_W1, l1_b1,
           l2_W0, l2_b0, l2_W1, l2_b1,
           output_bias):
    nbrT = jnp.transpose(neighbor_idx.astype(jnp.int32))
    nbrP = jnp.pad(nbrT, ((0, 0), (0, S - N)))
    idxc = jnp.transpose(nbrP.reshape(4, NC, CS), (1, 0, 2))
    idxc = idxc.reshape(NC, 1, MC)
    nid3d = node_ids.astype(jnp.int32).reshape(N // BN, 1, BN)
    embp = jnp.pad(emb, ((0, VP - emb.shape[0]), (0, 0)))

    params = [
        (l0_W0.astype(jnp.bfloat16), l0_b0.reshape(1, D),
         l0_W1.astype(jnp.bfloat16), l0_b1.reshape(1, D)),
        (l1_W0.astype(jnp.bfloat16), l1_b0.reshape(1, D),
         l1_W1.astype(jnp.bfloat16), l1_b1.reshape(1, D)),
        (l2_W0.astype(jnp.bfloat16), l2_b0.reshape(1, D),
         l2_W1.astype(jnp.bfloat16), l2_b1.reshape(1, D)),
    ]

    h = _tc_embed(nid3d, embp)
    for li, (W0, b0r, W1, b1r) in enumerate(params):
        msgs = [_sc_gather(h, idxc[c], MC) for c in range(NC)]
        if li < 2:
            hbuf = jnp.zeros((N, D), jnp.float32)
            for c in range(NC):
                hbuf = _tc_layer_chunk(c, msgs[c], W0, b0r, W1, b1r, hbuf)
            h = hbuf
        else:
            partials = [_tc_final_chunk(c, msgs[c], W0, b0r, W1, b1r)
                        for c in range(NC)]
    sums = sum(partials)
    half = jnp.float32(N // 2)
    logit = jnp.sum(sums[0] * sums[1], keepdims=True) / (half * half)
    return logit + output_bias


# speedup vs baseline: 1.0552x; 1.0008x over previous
import functools

import jax
import jax.numpy as jnp
from jax.experimental import pallas as pl
from jax.experimental.pallas import tpu as pltpu
from jax.experimental.pallas import tpu_sc as plsc

N = 100000
D = 128
VP = 40
NC = 8
CS = 12800
S = NC * CS
MC = 4 * CS
BN = 400
CB = CS // BN
HALF_BLOCK = (N // 2) // BN
GW = 128
NSTR = 2
GB = GW * NSTR

_CHUNK_REAL = [min(CS, N - c * CS) for c in range(NC)]


def _sc_gather(table, idx2d, m_rows):
    mesh = plsc.VectorSubcoreMesh(core_axis_name="c", subcore_axis_name="s")

    @functools.partial(
        pl.kernel,
        out_type=jax.ShapeDtypeStruct((m_rows, D), jnp.float32),
        mesh=mesh,
        scratch_types=[pltpu.SemaphoreType.DMA],
    )
    def gather_kernel(tab_hbm, idx_hbm, out_hbm, gsem):
        def body(i_vmem, o_vmem):
            for j in range(NSTR):
                pltpu.async_copy(
                    tab_hbm.at[i_vmem.at[0, pl.ds(j * GW, GW)]],
                    o_vmem.at[pl.ds(j * GW, GW), :],
                    gsem,
                )
            for j in range(NSTR):
                pltpu.make_async_copy(
                    tab_hbm.at[i_vmem.at[0, pl.ds(j * GW, GW)]],
                    o_vmem.at[pl.ds(j * GW, GW), :],
                    gsem,
                ).wait()

        pltpu.emit_pipeline(
            body,
            grid=(m_rows // GB,),
            in_specs=[pl.BlockSpec((1, GB), lambda i: (0, i))],
            out_specs=[pl.BlockSpec((GB, D), lambda i: (i, 0))],
            core_axis_name=("c", "s"),
            dimension_semantics=(pltpu.PARALLEL,),
        )(idx_hbm, out_hbm)

    return gather_kernel(table, idx2d)


def _tc_embed(nid3d, embp):
    def body(ids_ref, emb_ref, out):
        ids = ids_ref[0, 0, :]
        iota = jax.lax.broadcasted_iota(jnp.int32, (BN, VP), 1)
        oh = (ids.reshape(BN, 1) == iota).astype(jnp.float32)
        out[...] = jnp.dot(oh, emb_ref[...], preferred_element_type=jnp.float32)

    return pl.pallas_call(
        body,
        grid=(N // BN,),
        in_specs=[
            pl.BlockSpec((1, 1, BN), lambda i: (i, 0, 0)),
            pl.BlockSpec((VP, D), lambda i: (0, 0)),
        ],
        out_specs=pl.BlockSpec((BN, D), lambda i: (i, 0)),
        out_shape=jax.ShapeDtypeStruct((N, D), jnp.float32),
    )(nid3d, embp)


def _mlp_block(m0, m1, m2, m3, w0, b0r, w1, b1r):
    bf = jnp.bfloat16
    x = jnp.dot(m0[...].astype(bf), w0[0:128, :], preferred_element_type=jnp.float32)
    x = x + jnp.dot(m1[...].astype(bf), w0[128:256, :], preferred_element_type=jnp.float32)
    x = x + jnp.dot(m2[...].astype(bf), w0[256:384, :], preferred_element_type=jnp.float32)
    x = x + jnp.dot(m3[...].astype(bf), w0[384:512, :], preferred_element_type=jnp.float32)
    x = jnp.maximum(x + b0r[...], 0.0).astype(bf)
    return jnp.dot(x, w1[...], preferred_element_type=jnp.float32) + b1r[...]


def _chunk_in_specs(grid_c):
    del grid_c
    return [
        pl.BlockSpec((BN, D), (lambda i, k=k: (k * CB + i, 0)))
        for k in range(4)
    ] + [
        pl.BlockSpec((4 * D, D), lambda i: (0, 0)),
        pl.BlockSpec((1, D), lambda i: (0, 0)),
        pl.BlockSpec((D, D), lambda i: (0, 0)),
        pl.BlockSpec((1, D), lambda i: (0, 0)),
    ]


def _tc_layer_chunk(c, msgs_c, W0, b0_2d, W1, b1_2d, hbuf):
    grid_c = _CHUNK_REAL[c] // BN

    def body(m0, m1, m2, m3, w0, b0r, w1, b1r, _hb, out):
        out[...] = _mlp_block(m0, m1, m2, m3, w0, b0r, w1, b1r)

    return pl.pallas_call(
        body,
        grid=(grid_c,),
        in_specs=_chunk_in_specs(grid_c) + [
            pl.BlockSpec((BN, D), (lambda i: (c * CB + i, 0))),
        ],
        out_specs=pl.BlockSpec((BN, D), (lambda i: (c * CB + i, 0))),
        out_shape=jax.ShapeDtypeStruct((N, D), jnp.float32),
        input_output_aliases={8: 0},
    )(msgs_c, msgs_c, msgs_c, msgs_c, W0, b0_2d, W1, b1_2d, hbuf)


def _tc_final_chunk(c, msgs_c, W0, b0_2d, W1, b1_2d):
    grid_c = _CHUNK_REAL[c] // BN

    def body(m0, m1, m2, m3, w0, b0r, w1, b1r, out):
        i = pl.program_id(0)
        h = _mlp_block(m0, m1, m2, m3, w0, b0r, w1, b1r)
        colsum = jnp.sum(h, axis=0, keepdims=True)

        @pl.when(i == 0)
        def _():
            out[...] = jnp.zeros_like(out)

        @pl.when(c * CB + i < HALF_BLOCK)
        def _():
            out[0:1, :] = out[0:1, :] + colsum

        @pl.when(c * CB + i >= HALF_BLOCK)
        def _():
            out[1:2, :] = out[1:2, :] + colsum

    return pl.pallas_call(
        body,
        grid=(grid_c,),
        in_specs=_chunk_in_specs(grid_c),
        out_specs=pl.BlockSpec((2, D), lambda i: (0, 0)),
        out_shape=jax.ShapeDtypeStruct((2, D), jnp.float32),
    )(msgs_c, msgs_c, msgs_c, msgs_c, W0, b0_2d, W1, b1_2d)


def kernel(node_ids, neighbor_idx, emb,
           l0_W0, l0_b0, l0_W1, l0_b1,
           l1_W0, l1_b0, l1_W1, l1_b1,
           l2_W0, l2_b0, l2_W1, l2_b1,
           output_bias):
    nbrT = jnp.transpose(neighbor_idx.astype(jnp.int32))
    nbrP = jnp.pad(nbrT, ((0, 0), (0, S - N)))
    idxc = jnp.transpose(nbrP.reshape(4, NC, CS), (1, 0, 2))
    idxc = idxc.reshape(NC, 1, MC)
    nid3d = node_ids.astype(jnp.int32).reshape(N // BN, 1, BN)
    embp = jnp.pad(emb, ((0, VP - emb.shape[0]), (0, 0)))

    params = [
        (l0_W0.astype(jnp.bfloat16), l0_b0.reshape(1, D),
         l0_W1.astype(jnp.bfloat16), l0_b1.reshape(1, D)),
        (l1_W0.astype(jnp.bfloat16), l1_b0.reshape(1, D),
         l1_W1.astype(jnp.bfloat16), l1_b1.reshape(1, D)),
        (l2_W0.astype(jnp.bfloat16), l2_b0.reshape(1, D),
         l2_W1.astype(jnp.bfloat16), l2_b1.reshape(1, D)),
    ]

    h = _tc_embed(nid3d, embp)
    for li, (W0, b0r, W1, b1r) in enumerate(params):
        msgs = [_sc_gather(h, idxc[c], MC) for c in range(NC)]
        if li < 2:
            hbuf = jnp.zeros((N, D), jnp.float32)
            for c in range(NC):
                hbuf = _tc_layer_chunk(c, msgs[c], W0, b0r, W1, b1r, hbuf)
            h = hbuf
        else:
            partials = [_tc_final_chunk(c, msgs[c], W0, b0r, W1, b1r)
                        for c in range(NC)]
    sums = sum(partials)
    half = jnp.float32(N // 2)
    logit = jnp.sum(sums[0] * sums[1], keepdims=True) / (half * half)
    return logit + output_bias


# speedup vs baseline: 1.4396x; 1.3644x over previous
import dataclasses
import functools

import jax
import jax.numpy as jnp
from jax.experimental import pallas as pl
from jax.experimental.pallas import tpu as pltpu
from jax.experimental.pallas import tpu_sc as plsc


def _sc_compiler_params():
    cp = pltpu.CompilerParams()
    if "needs_layout_passes" in pltpu.CompilerParams.__dataclass_fields__:
        cp = dataclasses.replace(cp, needs_layout_passes=False)
    return cp

N = 100000
D = 128
VP = 40
NC = 8
CS = 12800
S = NC * CS
M4 = 4 * S
MC = 4 * CS
BN = 400
CB = CS // BN
HALF_BLOCK = (N // 2) // BN
GW = 128
NSTR = 2
GB = GW * NSTR
NW = 32
CW = M4 // NW
W2 = 1600

_CHUNK_REAL = [min(CS, N - c * CS) for c in range(NC)]


def _sc_cidx(nids, idxflat):
    mesh = plsc.VectorSubcoreMesh(core_axis_name="c", subcore_axis_name="s")

    @functools.partial(
        pl.kernel,
        out_type=jax.ShapeDtypeStruct((M4,), jnp.int32),
        mesh=mesh,
        scratch_types=[
            pltpu.VMEM((N,), jnp.int32),
            pltpu.VMEM((W2,), jnp.int32),
            pltpu.VMEM((W2,), jnp.int32),
        ],
        compiler_params=_sc_compiler_params(),
    )
    def cidx_kernel(nids_hbm, idx_hbm, out_hbm, nids_v, idx_v, out_v):
        wid = jax.lax.axis_index("s") * 2 + jax.lax.axis_index("c")
        base = wid * CW
        pltpu.sync_copy(nids_hbm, nids_v)

        @pl.loop(0, CW, step=W2)
        def _(g):
            pltpu.sync_copy(idx_hbm.at[pl.ds(base + g, W2)], idx_v)

            @pl.loop(0, W2, step=16)
            def _(i):
                iv = idx_v[pl.ds(i, 16)]
                out_v[pl.ds(i, 16)] = plsc.load_gather(nids_v, [iv])

            pltpu.sync_copy(out_v, out_hbm.at[pl.ds(base + g, W2)])

    return cidx_kernel(nids, idxflat)


def _sc_gather(table, idx2d, m_rows):
    mesh = plsc.VectorSubcoreMesh(core_axis_name="c", subcore_axis_name="s")

    @functools.partial(
        pl.kernel,
        out_type=jax.ShapeDtypeStruct((m_rows, D), jnp.float32),
        mesh=mesh,
        scratch_types=[pltpu.SemaphoreType.DMA],
    )
    def gather_kernel(tab_hbm, idx_hbm, out_hbm, gsem):
        def body(i_vmem, o_vmem):
            for j in range(NSTR):
                pltpu.async_copy(
                    tab_hbm.at[i_vmem.at[0, pl.ds(j * GW, GW)]],
                    o_vmem.at[pl.ds(j * GW, GW), :],
                    gsem,
                )
            for j in range(NSTR):
                pltpu.make_async_copy(
                    tab_hbm.at[i_vmem.at[0, pl.ds(j * GW, GW)]],
                    o_vmem.at[pl.ds(j * GW, GW), :],
                    gsem,
                ).wait()

        pltpu.emit_pipeline(
            body,
            grid=(m_rows // GB,),
            in_specs=[pl.BlockSpec((1, GB), lambda i: (0, i))],
            out_specs=[pl.BlockSpec((GB, D), lambda i: (i, 0))],
            core_axis_name=("c", "s"),
            dimension_semantics=(pltpu.PARALLEL,),
        )(idx_hbm, out_hbm)

    return gather_kernel(table, idx2d)


def _mlp_core(m0, m1, m2, m3, w0, b0r, w1, b1r):
    x = jnp.dot(m0, w0[0:128, :], preferred_element_type=jnp.float32)
    x = x + jnp.dot(m1, w0[128:256, :], preferred_element_type=jnp.float32)
    x = x + jnp.dot(m2, w0[256:384, :], preferred_element_type=jnp.float32)
    x = x + jnp.dot(m3, w0[384:512, :], preferred_element_type=jnp.float32)
    x = jnp.maximum(x + b0r, 0.0).astype(jnp.bfloat16)
    return jnp.dot(x, w1, preferred_element_type=jnp.float32) + b1r


def _tc_layer1(cidx3d, embb, W0, b0_2d, W1, b1_2d):
    def body(c0, c1, c2, c3, emb_ref, w0, b0r, w1, b1r, out):
        iota = jax.lax.broadcasted_iota(jnp.int32, (BN, VP), 1)
        ms = []
        for cref in (c0, c1, c2, c3):
            oh = (cref[0, 0, :].reshape(BN, 1) == iota).astype(jnp.bfloat16)
            ms.append(jnp.dot(oh, emb_ref[...],
                              preferred_element_type=jnp.float32)
                      .astype(jnp.bfloat16))
        out[...] = _mlp_core(ms[0], ms[1], ms[2], ms[3],
                             w0[...], b0r[...], w1[...], b1r[...])

    def slot_spec(k):
        return pl.BlockSpec(
            (1, 1, BN),
            (lambda j, k=k: ((j // CB) * 4 * CB + k * CB + j % CB, 0, 0)),
        )

    return pl.pallas_call(
        body,
        grid=(N // BN,),
        in_specs=[slot_spec(k) for k in range(4)] + [
            pl.BlockSpec((VP, D), lambda j: (0, 0)),
            pl.BlockSpec((4 * D, D), lambda j: (0, 0)),
            pl.BlockSpec((1, D), lambda j: (0, 0)),
            pl.BlockSpec((D, D), lambda j: (0, 0)),
            pl.BlockSpec((1, D), lambda j: (0, 0)),
        ],
        out_specs=pl.BlockSpec((BN, D), lambda j: (j, 0)),
        out_shape=jax.ShapeDtypeStruct((N, D), jnp.float32),
    )(cidx3d, cidx3d, cidx3d, cidx3d, embb, W0, b0_2d, W1, b1_2d)


def _chunk_in_specs():
    return [
        pl.BlockSpec((BN, D), (lambda i, k=k: (k * CB + i, 0)))
        for k in range(4)
    ] + [
        pl.BlockSpec((4 * D, D), lambda i: (0, 0)),
        pl.BlockSpec((1, D), lambda i: (0, 0)),
        pl.BlockSpec((D, D), lambda i: (0, 0)),
        pl.BlockSpec((1, D), lambda i: (0, 0)),
    ]


def _tc_layer_chunk(c, msgs_c, W0, b0_2d, W1, b1_2d, hbuf):
    grid_c = _CHUNK_REAL[c] // BN

    def body(m0, m1, m2, m3, w0, b0r, w1, b1r, _hb, out):
        bf = jnp.bfloat16
        out[...] = _mlp_core(
            m0[...].astype(bf), m1[...].astype(bf),
            m2[...].astype(bf), m3[...].astype(bf),
            w0[...], b0r[...], w1[...], b1r[...])

    return pl.pallas_call(
        body,
        grid=(grid_c,),
        in_specs=_chunk_in_specs() + [
            pl.BlockSpec((BN, D), (lambda i: (c * CB + i, 0))),
        ],
        out_specs=pl.BlockSpec((BN, D), (lambda i: (c * CB + i, 0))),
        out_shape=jax.ShapeDtypeStruct((N, D), jnp.float32),
        input_output_aliases={8: 0},
    )(msgs_c, msgs_c, msgs_c, msgs_c, W0, b0_2d, W1, b1_2d, hbuf)


def _tc_final_chunk(c, msgs_c, W0, b0_2d, W1, b1_2d):
    grid_c = _CHUNK_REAL[c] // BN

    def body(m0, m1, m2, m3, w0, b0r, w1, b1r, out):
        i = pl.program_id(0)
        bf = jnp.bfloat16
        h = _mlp_core(
            m0[...].astype(bf), m1[...].astype(bf),
            m2[...].astype(bf), m3[...].astype(bf),
            w0[...], b0r[...], w1[...], b1r[...])
        colsum = jnp.sum(h, axis=0, keepdims=True)

        @pl.when(i == 0)
        def _():
            out[...] = jnp.zeros_like(out)

        @pl.when(c * CB + i < HALF_BLOCK)
        def _():
            out[0:1, :] = out[0:1, :] + colsum

        @pl.when(c * CB + i >= HALF_BLOCK)
        def _():
            out[1:2, :] = out[1:2, :] + colsum

    return pl.pallas_call(
        body,
        grid=(grid_c,),
        in_specs=_chunk_in_specs(),
        out_specs=pl.BlockSpec((2, D), lambda i: (0, 0)),
        out_shape=jax.ShapeDtypeStruct((2, D), jnp.float32),
    )(msgs_c, msgs_c, msgs_c, msgs_c, W0, b0_2d, W1, b1_2d)


def kernel(node_ids, neighbor_idx, emb,
           l0_W0, l0_b0, l0_W1, l0_b1,
           l1_W0, l1_b0, l1_W1, l1_b1,
           l2_W0, l2_b0, l2_W1, l2_b1,
           output_bias):
    nbrT = jnp.transpose(neighbor_idx.astype(jnp.int32))
    nbrP = jnp.pad(nbrT, ((0, 0), (0, S - N)))
    idxc = jnp.transpose(nbrP.reshape(4, NC, CS), (1, 0, 2))
    idxflat = idxc.reshape(M4)
    idxc = idxc.reshape(NC, 1, MC)
    embb = jnp.pad(emb, ((0, VP - emb.shape[0]), (0, 0))).astype(jnp.bfloat16)

    params = [
        (l0_W0.astype(jnp.bfloat16), l0_b0.reshape(1, D),
         l0_W1.astype(jnp.bfloat16), l0_b1.reshape(1, D)),
        (l1_W0.astype(jnp.bfloat16), l1_b0.reshape(1, D),
         l1_W1.astype(jnp.bfloat16), l1_b1.reshape(1, D)),
        (l2_W0.astype(jnp.bfloat16), l2_b0.reshape(1, D),
         l2_W1.astype(jnp.bfloat16), l2_b1.reshape(1, D)),
    ]

    cidx = _sc_cidx(node_ids.astype(jnp.int32), idxflat)
    cidx3d = cidx.reshape(M4 // BN, 1, BN)
    W0, b0r, W1, b1r = params[0]
    h = _tc_layer1(cidx3d, embb, W0, b0r, W1, b1r)

    for li in (1, 2):
        W0, b0r, W1, b1r = params[li]
        msgs = [_sc_gather(h, idxc[c], MC) for c in range(NC)]
        if li < 2:
            hbuf = jnp.zeros((N, D), jnp.float32)
            for c in range(NC):
                hbuf = _tc_layer_chunk(c, msgs[c], W0, b0r, W1, b1r, hbuf)
            h = hbuf
        else:
            partials = [_tc_final_chunk(c, msgs[c], W0, b0r, W1, b1r)
                        for c in range(NC)]
    sums = sum(partials)
    half = jnp.float32(N // 2)
    logit = jnp.sum(sums[0] * sums[1], keepdims=True) / (half * half)
    return logit + output_bias
